# baseline (speedup 1.0000x reference)
import numpy as np
import jax
import jax.numpy as jnp
from jax import lax
from jax.experimental import pallas as pl
from jax.experimental.pallas import tpu as pltpu
from jax.experimental.pallas import tpu_sc as plsc

_IDX_PAD = 216
_LANES = 16


def _build_index_map():
    def g(src, dim, idx):
        dim = dim % src.ndim
        sl = tuple(
            slice(None) if a == dim else slice(0, idx.shape[a])
            for a in range(src.ndim)
        )
        return np.take_along_axis(src[sl], idx, axis=dim)

    bx = np.arange(12)
    by = np.arange(28).reshape(4, 7)
    bz = np.arange(24).reshape(2, 3, 4)
    bd = np.arange(625).reshape(5, 5, 5, 5)

    ix = np.array([7, 9, 11])
    iy0 = np.array([[1, 3, 2], [0, 3, 1]])
    iy1 = np.array([[1, 3, 2, 4, 6, 5], [4, 3, 2, 1, 5, 6]])
    iz0 = np.array([[[0], [1], [0]], [[1], [0], [1]]])
    iz1 = np.array([[[0], [1], [2]], [[1], [2], [0]]])
    iz2 = np.array([[[0, 1, 2, 3]], [[2, 1, 0, 3]]])
    zz = np.array([[[[0, 1, 0, 1, 0], [1, 0, 1, 0, 1],
                     [0, 1, 0, 1, 0], [1, 0, 1, 0, 1]]],
                   [[[1, 0, 3, 4, 1], [0, 1, 0, 1, 0],
                     [1, 0, 1, 0, 1], [0, 1, 0, 1, 0]]]])

    parts = [
        (0, g(bx, 0, ix)),
        (1, g(by, 0, iy0)),
        (1, g(by, 1, iy1)),
        (2, g(bz, -3, iz0)),
        (2, g(bz, -2, iz1)),
        (2, g(bz, -1, iz2)),
        (3, g(bd, 0, zz)),
        (3, g(bd, 1, zz)),
        (3, g(bd, 2, zz)),
        (3, g(bd, 3, zz)),
    ]
    table = np.zeros(_IDX_PAD, dtype=np.int32)
    specs = []
    off = 0
    for src_slot, p in parts:
        table[off:off + p.size] = p.ravel()
        specs.append((src_slot, off, p.shape))
        off += -(-p.size // 8) * 8
    assert off <= _IDX_PAD
    return table, specs


_IDX_NP, _OUT_SPECS = _build_index_map()

_GATHER_RUNS = ((0, 0, 8), (1, 8, 24), (2, 32, 24), (3, 56, 160))


_SRC_OFFSETS = (0, 12, 40, 64)
_IDX_GLOBAL_NP = _IDX_NP.copy()
for _slot, _off, _ln in _GATHER_RUNS:
    _IDX_GLOBAL_NP[_off:_off + _ln] += _SRC_OFFSETS[_slot]


def _gather_body(flat_hbm, idx_hbm, out_hbm, idx_v, out_v, sem):
    wid = lax.axis_index("s")

    @pl.when(wid == 0)
    def _():
        pltpu.sync_copy(idx_hbm, idx_v)
        pltpu.async_copy(flat_hbm.at[idx_v], out_v, sem).wait()
        pltpu.sync_copy(out_v, out_hbm)


def kernel(x, y, z, d):
    mesh = plsc.VectorSubcoreMesh(
        core_axis_name="c", subcore_axis_name="s", num_cores=1
    )
    out_flat = pl.kernel(
        _gather_body,
        mesh=mesh,
        out_type=jax.ShapeDtypeStruct((_IDX_PAD,), jnp.float32),
        scratch_types=[
            pltpu.VMEM((_IDX_PAD,), jnp.int32),
            pltpu.VMEM((_IDX_PAD,), jnp.float32),
            pltpu.SemaphoreType.DMA,
        ],
    )(
        jnp.concatenate([x.ravel(), y.ravel(), z.ravel(), d.ravel()]),
        jnp.asarray(_IDX_GLOBAL_NP),
    )

    return tuple(
        out_flat[off:off + int(np.prod(shape))].reshape(shape)
        for _, off, shape in _OUT_SPECS
    )


# speedup vs baseline: 1.0056x; 1.0056x over previous
import numpy as np
import jax
import jax.numpy as jnp
from jax import lax
from jax.experimental import pallas as pl
from jax.experimental.pallas import tpu as pltpu
from jax.experimental.pallas import tpu_sc as plsc

_IDX_PAD = 216


def _build_index_map():
    def g(src, dim, idx):
        dim = dim % src.ndim
        sl = tuple(
            slice(None) if a == dim else slice(0, idx.shape[a])
            for a in range(src.ndim)
        )
        return np.take_along_axis(src[sl], idx, axis=dim)

    bx = np.arange(12)
    by = np.arange(28).reshape(4, 7)
    bz = np.arange(24).reshape(2, 3, 4)
    bd = np.arange(625).reshape(5, 5, 5, 5)

    ix = np.array([7, 9, 11])
    iy0 = np.array([[1, 3, 2], [0, 3, 1]])
    iy1 = np.array([[1, 3, 2, 4, 6, 5], [4, 3, 2, 1, 5, 6]])
    iz0 = np.array([[[0], [1], [0]], [[1], [0], [1]]])
    iz1 = np.array([[[0], [1], [2]], [[1], [2], [0]]])
    iz2 = np.array([[[0, 1, 2, 3]], [[2, 1, 0, 3]]])
    zz = np.array([[[[0, 1, 0, 1, 0], [1, 0, 1, 0, 1],
                     [0, 1, 0, 1, 0], [1, 0, 1, 0, 1]]],
                   [[[1, 0, 3, 4, 1], [0, 1, 0, 1, 0],
                     [1, 0, 1, 0, 1], [0, 1, 0, 1, 0]]]])

    parts = [
        (0, g(bx, 0, ix)),
        (1, g(by, 0, iy0)),
        (1, g(by, 1, iy1)),
        (2, g(bz, -3, iz0)),
        (2, g(bz, -2, iz1)),
        (2, g(bz, -1, iz2)),
        (3, g(bd, 0, zz)),
        (3, g(bd, 1, zz)),
        (3, g(bd, 2, zz)),
        (3, g(bd, 3, zz)),
    ]
    table = np.zeros(_IDX_PAD, dtype=np.int32)
    specs = []
    off = 0
    for src_slot, p in parts:
        table[off:off + p.size] = p.ravel()
        specs.append((src_slot, off, p.shape))
        off += -(-p.size // 8) * 8
    assert off <= _IDX_PAD
    return table, specs


_IDX_NP, _OUT_SPECS = _build_index_map()

_GATHER_RUNS = ((0, 0, 8), (1, 8, 24), (2, 32, 24), (3, 56, 160))


_SRC_OFFSETS = (0, 12, 40, 64)
_IDX_GLOBAL_NP = _IDX_NP.copy()
for _slot, _off, _ln in _GATHER_RUNS:
    _IDX_GLOBAL_NP[_off:_off + _ln] += _SRC_OFFSETS[_slot]


def _gather_body(flat_hbm, idx_hbm, out_hbm, idx_v, out_v, sem):
    wid = lax.axis_index("s")

    @pl.when(wid == 0)
    def _():
        pltpu.sync_copy(idx_hbm, idx_v)
        pltpu.async_copy(flat_hbm.at[idx_v], out_v, sem).wait()
        pltpu.sync_copy(out_v, out_hbm)


def kernel(x, y, z, d):
    mesh = plsc.VectorSubcoreMesh(
        core_axis_name="c", subcore_axis_name="s", num_cores=1
    )
    out_flat = pl.kernel(
        _gather_body,
        mesh=mesh,
        out_type=jax.ShapeDtypeStruct((_IDX_PAD,), jnp.float32),
        scratch_types=[
            pltpu.VMEM((_IDX_PAD,), jnp.int32),
            pltpu.VMEM((_IDX_PAD,), jnp.float32),
            pltpu.SemaphoreType.DMA,
        ],
    )(
        jnp.concatenate([x.ravel(), y.ravel(), z.ravel(), d.ravel()]),
        jnp.asarray(_IDX_GLOBAL_NP),
    )

    return tuple(
        out_flat[off:off + int(np.prod(shape))].reshape(shape)
        for _, off, shape in _OUT_SPECS
    )
